# baseline (speedup 1.0000x reference)
import dataclasses

import jax
import jax.numpy as jnp
from jax import lax
from jax.experimental import pallas as pl
from jax.experimental.pallas import tpu as pltpu
from jax.experimental.pallas import tpu_sc as plsc

NC, NS, L = 2, 16, 16
NW = NC * NS
VOCAB = 1000000
B = 4096
D = 64
N_POS = 20
N_CTX = 70
N_PAD = 72
B_PER_W = B // NW
CHUNK = 4
N_CHUNKS = B_PER_W // CHUNK
CB = 8192


HALF = 524288


def _relayout_body(a_ref, b_ref, o_ref):
    o_ref[:, 0:D] = a_ref[...].T
    o_ref[:, D:2 * D] = b_ref[...].T


def _sc_body(ctx_hbm, vt_hbm, w1_hbm, scores_hbm,
             v_cols, idx_c0, idx_c1, par0, par1,
             u_buf0, u_buf1, score_v, sem0, sem1):
    wid = lax.axis_index("s") * NC + lax.axis_index("c")
    base = wid * B_PER_W

    pltpu.sync_copy(vt_hbm.at[:, pl.ds(base, B_PER_W)], v_cols)

    lane = lax.iota(jnp.int32, L)
    m_last = lane == (L - 1)

    def stage_and_fire(cb, ibuf, pbuf, ubuf, s):
        row0 = base + cb * CHUNK
        pltpu.sync_copy(ctx_hbm.at[pl.ds(row0 * N_PAD, CHUNK * N_PAD)], ibuf)
        for g in range(CHUNK * N_PAD // L):
            iv = ibuf[pl.ds(g * L, L)]
            ibuf[pl.ds(g * L, L)] = iv & (HALF - 1)
            pbuf[pl.ds(g * L, L)] = (iv >> 19) << 6
        for j in range(CHUNK):
            pltpu.make_async_copy(
                w1_hbm.at[ibuf.at[pl.ds(j * N_PAD, N_PAD)]],
                ubuf.at[pl.ds(j * N_PAD, N_PAD)], s).start()

    def drain(ubuf, s):
        for j in range(CHUNK):
            pltpu.make_async_copy(
                w1_hbm.at[idx_c0.at[pl.ds(j * N_PAD, N_PAD)]],
                ubuf.at[pl.ds(j * N_PAD, N_PAD)], s).wait()

    def compute(cb, pbuf, ubuf):
        for j in range(CHUNK):
            col = jnp.full((L,), cb * CHUNK + j, jnp.int32)
            v0 = plsc.load_gather(v_cols, [lane, col])
            v1 = plsc.load_gather(v_cols, [lane + L, col])
            v2 = plsc.load_gather(v_cols, [lane + 2 * L, col])
            v3 = plsc.load_gather(v_cols, [lane + 3 * L, col])

            @plsc.parallel_loop(0, N_CTX, unroll=8)
            def _(n):
                urow = ubuf.at[j * N_PAD + n]
                off = plsc.load_gather(
                    pbuf, [jnp.full((L,), j * N_PAD + n, jnp.int32)]) + lane
                acc = ((plsc.load_gather(urow, [off]) * v0
                        + plsc.load_gather(urow, [off + L]) * v1)
                       + (plsc.load_gather(urow, [off + 2 * L]) * v2
                          + plsc.load_gather(urow, [off + 3 * L]) * v3))
                tot = plsc.cumsum(acc)
                plsc.store_scatter(score_v,
                                   [jnp.full((L,), n, jnp.int32), col],
                                   tot, mask=m_last)

    stage_and_fire(0, idx_c0, par0, u_buf0, sem0)

    @pl.loop(0, N_CHUNKS // 2)
    def _(i):
        c = 2 * i
        drain(u_buf0, sem0)
        stage_and_fire(c + 1, idx_c1, par1, u_buf1, sem1)
        compute(c, par0, u_buf0)
        drain(u_buf1, sem1)

        @pl.when(c + 2 < N_CHUNKS)
        def _():
            stage_and_fire(c + 2, idx_c0, par0, u_buf0, sem0)

        compute(c + 1, par1, u_buf1)

    pltpu.sync_copy(score_v, scores_hbm.at[:, pl.ds(base, B_PER_W)])


def kernel(target, pos_context, neg_context, in_W, out_W):
    ctx = jnp.concatenate(
        [pos_context, neg_context, pos_context[:, : N_PAD - N_CTX]], axis=1)
    ctx_flat = ctx.astype(jnp.int32).reshape(-1)
    tgt = target.astype(jnp.int32)

    v_t = jnp.take(in_W, tgt, axis=0).T

    n_in_blocks = (VOCAB + CB - 1) // CB
    grid = HALF // CB
    w1 = pl.pallas_call(
        _relayout_body,
        grid=(grid,),
        in_specs=[pl.BlockSpec((D, CB), lambda i: (0, i)),
                  pl.BlockSpec((D, CB),
                               lambda i: (0, jnp.minimum(i + HALF // CB,
                                                         n_in_blocks - 1)))],
        out_specs=pl.BlockSpec((CB, 2 * D), lambda i: (i, 0)),
        out_shape=jax.ShapeDtypeStruct((HALF, 2 * D), jnp.float32),
        compiler_params=pltpu.CompilerParams(
            dimension_semantics=("parallel",)),
    )(out_W.T, out_W.T)

    mesh = plsc.VectorSubcoreMesh(core_axis_name="c", subcore_axis_name="s",
                                  num_cores=NC, num_subcores=NS)
    cp = pltpu.CompilerParams()
    if "needs_layout_passes" in pltpu.CompilerParams.__dataclass_fields__:
        cp = dataclasses.replace(cp, needs_layout_passes=False)
    scores_t = pl.kernel(
        _sc_body,
        out_type=jax.ShapeDtypeStruct((N_PAD, B), jnp.float32),
        mesh=mesh,
        compiler_params=cp,
        scratch_types=[
            pltpu.VMEM((D, B_PER_W), jnp.float32),
            pltpu.VMEM((CHUNK * N_PAD,), jnp.int32),
            pltpu.VMEM((CHUNK * N_PAD,), jnp.int32),
            pltpu.VMEM((CHUNK * N_PAD,), jnp.int32),
            pltpu.VMEM((CHUNK * N_PAD,), jnp.int32),
            pltpu.VMEM((CHUNK * N_PAD, 2 * D), jnp.float32),
            pltpu.VMEM((CHUNK * N_PAD, 2 * D), jnp.float32),
            pltpu.VMEM((N_PAD, B_PER_W), jnp.float32),
            pltpu.SemaphoreType.DMA,
            pltpu.SemaphoreType.DMA,
        ],
    )(ctx_flat, v_t, w1)

    return scores_t[:N_POS].T, scores_t[N_POS:N_CTX].T


# speedup vs baseline: 1.0027x; 1.0027x over previous
import dataclasses

import jax
import jax.numpy as jnp
from jax import lax
from jax.experimental import pallas as pl
from jax.experimental.pallas import tpu as pltpu
from jax.experimental.pallas import tpu_sc as plsc

NC, NS, L = 2, 16, 16
NW = NC * NS
VOCAB = 1000000
B = 4096
D = 64
N_POS = 20
N_CTX = 70
N_PAD = 72
B_PER_W = B // NW
CHUNK = 4
N_CHUNKS = B_PER_W // CHUNK
CB = 8192


HALF = 524288


def _relayout_body(a_ref, b_ref, o_ref):
    o_ref[:, 0:D] = a_ref[...].T
    o_ref[:, D:2 * D] = b_ref[...].T


def _sc_body(ctx_hbm, vt_hbm, w1_hbm, scores_hbm,
             v_cols, idx_c0, idx_c1, par0, par1,
             u_buf0, u_buf1, score_v, sem0, sem1):
    wid = lax.axis_index("s") * NC + lax.axis_index("c")
    base = wid * B_PER_W

    pltpu.sync_copy(vt_hbm.at[:, pl.ds(base, B_PER_W)], v_cols)

    lane = lax.iota(jnp.int32, L)
    m_last = lane == (L - 1)

    def stage_and_fire(cb, ibuf, pbuf, ubuf, s):
        row0 = base + cb * CHUNK
        pltpu.sync_copy(ctx_hbm.at[pl.ds(row0 * N_PAD, CHUNK * N_PAD)], ibuf)
        for g in range(CHUNK * N_PAD // L):
            iv = ibuf[pl.ds(g * L, L)]
            ibuf[pl.ds(g * L, L)] = iv & (HALF - 1)
            pbuf[pl.ds(g * L, L)] = (iv >> 19) << 6
        for j in range(CHUNK):
            pltpu.make_async_copy(
                w1_hbm.at[ibuf.at[pl.ds(j * N_PAD, N_PAD)]],
                ubuf.at[pl.ds(j * N_PAD, N_PAD)], s).start()

    def drain(ubuf, s):
        for j in range(CHUNK):
            pltpu.make_async_copy(
                w1_hbm.at[idx_c0.at[pl.ds(j * N_PAD, N_PAD)]],
                ubuf.at[pl.ds(j * N_PAD, N_PAD)], s).wait()

    def compute(cb, pbuf, ubuf):
        for j in range(CHUNK):
            col = jnp.full((L,), cb * CHUNK + j, jnp.int32)
            v0 = plsc.load_gather(v_cols, [lane, col])
            v1 = plsc.load_gather(v_cols, [lane + L, col])
            v2 = plsc.load_gather(v_cols, [lane + 2 * L, col])
            v3 = plsc.load_gather(v_cols, [lane + 3 * L, col])

            @plsc.parallel_loop(0, N_CTX, unroll=4)
            def _(n):
                urow = ubuf.at[j * N_PAD + n]
                off = plsc.load_gather(
                    pbuf, [jnp.full((L,), j * N_PAD + n, jnp.int32)]) + lane
                acc = ((plsc.load_gather(urow, [off]) * v0
                        + plsc.load_gather(urow, [off + L]) * v1)
                       + (plsc.load_gather(urow, [off + 2 * L]) * v2
                          + plsc.load_gather(urow, [off + 3 * L]) * v3))
                tot = plsc.cumsum(acc)
                plsc.store_scatter(score_v,
                                   [jnp.full((L,), n, jnp.int32), col],
                                   tot, mask=m_last)

    stage_and_fire(0, idx_c0, par0, u_buf0, sem0)

    @pl.loop(0, N_CHUNKS // 2)
    def _(i):
        c = 2 * i
        drain(u_buf0, sem0)
        stage_and_fire(c + 1, idx_c1, par1, u_buf1, sem1)
        compute(c, par0, u_buf0)
        drain(u_buf1, sem1)

        @pl.when(c + 2 < N_CHUNKS)
        def _():
            stage_and_fire(c + 2, idx_c0, par0, u_buf0, sem0)

        compute(c + 1, par1, u_buf1)

    pltpu.sync_copy(score_v, scores_hbm.at[:, pl.ds(base, B_PER_W)])


def kernel(target, pos_context, neg_context, in_W, out_W):
    ctx = jnp.concatenate(
        [pos_context, neg_context, pos_context[:, : N_PAD - N_CTX]], axis=1)
    ctx_flat = ctx.astype(jnp.int32).reshape(-1)
    tgt = target.astype(jnp.int32)

    v_t = jnp.take(in_W, tgt, axis=0).T

    n_in_blocks = (VOCAB + CB - 1) // CB
    grid = HALF // CB
    w1 = pl.pallas_call(
        _relayout_body,
        grid=(grid,),
        in_specs=[pl.BlockSpec((D, CB), lambda i: (0, i)),
                  pl.BlockSpec((D, CB),
                               lambda i: (0, jnp.minimum(i + HALF // CB,
                                                         n_in_blocks - 1)))],
        out_specs=pl.BlockSpec((CB, 2 * D), lambda i: (i, 0)),
        out_shape=jax.ShapeDtypeStruct((HALF, 2 * D), jnp.float32),
        compiler_params=pltpu.CompilerParams(
            dimension_semantics=("parallel",)),
    )(out_W.T, out_W.T)

    mesh = plsc.VectorSubcoreMesh(core_axis_name="c", subcore_axis_name="s",
                                  num_cores=NC, num_subcores=NS)
    cp = pltpu.CompilerParams()
    if "needs_layout_passes" in pltpu.CompilerParams.__dataclass_fields__:
        cp = dataclasses.replace(cp, needs_layout_passes=False)
    scores_t = pl.kernel(
        _sc_body,
        out_type=jax.ShapeDtypeStruct((N_PAD, B), jnp.float32),
        mesh=mesh,
        compiler_params=cp,
        scratch_types=[
            pltpu.VMEM((D, B_PER_W), jnp.float32),
            pltpu.VMEM((CHUNK * N_PAD,), jnp.int32),
            pltpu.VMEM((CHUNK * N_PAD,), jnp.int32),
            pltpu.VMEM((CHUNK * N_PAD,), jnp.int32),
            pltpu.VMEM((CHUNK * N_PAD,), jnp.int32),
            pltpu.VMEM((CHUNK * N_PAD, 2 * D), jnp.float32),
            pltpu.VMEM((CHUNK * N_PAD, 2 * D), jnp.float32),
            pltpu.VMEM((N_PAD, B_PER_W), jnp.float32),
            pltpu.SemaphoreType.DMA,
            pltpu.SemaphoreType.DMA,
        ],
    )(ctx_flat, v_t, w1)

    return scores_t[:N_POS].T, scores_t[N_POS:N_CTX].T


# speedup vs baseline: 1.0070x; 1.0043x over previous
import dataclasses

import jax
import jax.numpy as jnp
from jax import lax
from jax.experimental import pallas as pl
from jax.experimental.pallas import tpu as pltpu
from jax.experimental.pallas import tpu_sc as plsc

NC, NS, L = 2, 16, 16
NW = NC * NS
VOCAB = 1000000
B = 4096
D = 64
N_POS = 20
N_CTX = 70
N_PAD = 72
B_PER_W = B // NW
CHUNK = 4
N_CHUNKS = B_PER_W // CHUNK
CB = 16384


HALF = 524288


def _relayout_body(a_ref, b_ref, o_ref):
    o_ref[:, 0:D] = a_ref[...].T
    o_ref[:, D:2 * D] = b_ref[...].T


def _sc_body(ctx_hbm, vt_hbm, w1_hbm, scores_hbm,
             v_cols, idx_c0, idx_c1, par0, par1,
             u_buf0, u_buf1, score_v, sem0, sem1):
    wid = lax.axis_index("s") * NC + lax.axis_index("c")
    base = wid * B_PER_W

    pltpu.sync_copy(vt_hbm.at[:, pl.ds(base, B_PER_W)], v_cols)

    lane = lax.iota(jnp.int32, L)
    m_last = lane == (L - 1)

    def stage_and_fire(cb, ibuf, pbuf, ubuf, s):
        row0 = base + cb * CHUNK
        pltpu.sync_copy(ctx_hbm.at[pl.ds(row0 * N_PAD, CHUNK * N_PAD)], ibuf)
        for g in range(CHUNK * N_PAD // L):
            iv = ibuf[pl.ds(g * L, L)]
            ibuf[pl.ds(g * L, L)] = iv & (HALF - 1)
            pbuf[pl.ds(g * L, L)] = (iv >> 19) << 6
        for j in range(CHUNK):
            pltpu.make_async_copy(
                w1_hbm.at[ibuf.at[pl.ds(j * N_PAD, N_PAD)]],
                ubuf.at[pl.ds(j * N_PAD, N_PAD)], s).start()

    def drain(ubuf, s):
        for j in range(CHUNK):
            pltpu.make_async_copy(
                w1_hbm.at[idx_c0.at[pl.ds(j * N_PAD, N_PAD)]],
                ubuf.at[pl.ds(j * N_PAD, N_PAD)], s).wait()

    def compute(cb, pbuf, ubuf):
        for j in range(CHUNK):
            col = jnp.full((L,), cb * CHUNK + j, jnp.int32)
            v0 = plsc.load_gather(v_cols, [lane, col])
            v1 = plsc.load_gather(v_cols, [lane + L, col])
            v2 = plsc.load_gather(v_cols, [lane + 2 * L, col])
            v3 = plsc.load_gather(v_cols, [lane + 3 * L, col])

            @plsc.parallel_loop(0, N_CTX, unroll=4)
            def _(n):
                urow = ubuf.at[j * N_PAD + n]
                off = plsc.load_gather(
                    pbuf, [jnp.full((L,), j * N_PAD + n, jnp.int32)]) + lane
                acc = ((plsc.load_gather(urow, [off]) * v0
                        + plsc.load_gather(urow, [off + L]) * v1)
                       + (plsc.load_gather(urow, [off + 2 * L]) * v2
                          + plsc.load_gather(urow, [off + 3 * L]) * v3))
                tot = plsc.cumsum(acc)
                plsc.store_scatter(score_v,
                                   [jnp.full((L,), n, jnp.int32), col],
                                   tot, mask=m_last)

    stage_and_fire(0, idx_c0, par0, u_buf0, sem0)

    @pl.loop(0, N_CHUNKS // 2)
    def _(i):
        c = 2 * i
        drain(u_buf0, sem0)
        stage_and_fire(c + 1, idx_c1, par1, u_buf1, sem1)
        compute(c, par0, u_buf0)
        drain(u_buf1, sem1)

        @pl.when(c + 2 < N_CHUNKS)
        def _():
            stage_and_fire(c + 2, idx_c0, par0, u_buf0, sem0)

        compute(c + 1, par1, u_buf1)

    pltpu.sync_copy(score_v, scores_hbm.at[:, pl.ds(base, B_PER_W)])


def kernel(target, pos_context, neg_context, in_W, out_W):
    ctx = jnp.concatenate(
        [pos_context, neg_context, pos_context[:, : N_PAD - N_CTX]], axis=1)
    ctx_flat = ctx.astype(jnp.int32).reshape(-1)
    tgt = target.astype(jnp.int32)

    v_t = jnp.take(in_W, tgt, axis=0).T

    n_in_blocks = (VOCAB + CB - 1) // CB
    grid = HALF // CB
    w1 = pl.pallas_call(
        _relayout_body,
        grid=(grid,),
        in_specs=[pl.BlockSpec((D, CB), lambda i: (0, i)),
                  pl.BlockSpec((D, CB),
                               lambda i: (0, jnp.minimum(i + HALF // CB,
                                                         n_in_blocks - 1)))],
        out_specs=pl.BlockSpec((CB, 2 * D), lambda i: (i, 0)),
        out_shape=jax.ShapeDtypeStruct((HALF, 2 * D), jnp.float32),
        compiler_params=pltpu.CompilerParams(
            dimension_semantics=("parallel",)),
    )(out_W.T, out_W.T)

    mesh = plsc.VectorSubcoreMesh(core_axis_name="c", subcore_axis_name="s",
                                  num_cores=NC, num_subcores=NS)
    cp = pltpu.CompilerParams()
    if "needs_layout_passes" in pltpu.CompilerParams.__dataclass_fields__:
        cp = dataclasses.replace(cp, needs_layout_passes=False)
    scores_t = pl.kernel(
        _sc_body,
        out_type=jax.ShapeDtypeStruct((N_PAD, B), jnp.float32),
        mesh=mesh,
        compiler_params=cp,
        scratch_types=[
            pltpu.VMEM((D, B_PER_W), jnp.float32),
            pltpu.VMEM((CHUNK * N_PAD,), jnp.int32),
            pltpu.VMEM((CHUNK * N_PAD,), jnp.int32),
            pltpu.VMEM((CHUNK * N_PAD,), jnp.int32),
            pltpu.VMEM((CHUNK * N_PAD,), jnp.int32),
            pltpu.VMEM((CHUNK * N_PAD, 2 * D), jnp.float32),
            pltpu.VMEM((CHUNK * N_PAD, 2 * D), jnp.float32),
            pltpu.VMEM((N_PAD, B_PER_W), jnp.float32),
            pltpu.SemaphoreType.DMA,
            pltpu.SemaphoreType.DMA,
        ],
    )(ctx_flat, v_t, w1)

    return scores_t[:N_POS].T, scores_t[N_POS:N_CTX].T
